# baseline (speedup 1.0000x reference)
import functools

import jax
import jax.numpy as jnp
import numpy as np
from jax import lax
from jax.experimental import pallas as pl
from jax.experimental.pallas import tpu as pltpu
from jax.experimental.pallas import tpu_sc as plsc

N = 10000
E = 320000
D = 128
C = 40

NC = 2
NS = 16
L = 16
D2 = D // NC
DP = D2 // 2
EPS = E // NS
K = 80
NBUF = 8
CHUNKS = 256
EPP = CHUNKS * K
RPW = 624
TAIL = N - NS * RPW
ZR = 48

_PERM = np.concatenate([np.arange(0, 32, 2), np.arange(1, 32, 2),
                        32 + np.arange(0, 32, 2), 33 + np.arange(0, 32, 2)])
_PERM128 = np.concatenate([_PERM, 64 + _PERM])

_mesh = plsc.VectorSubcoreMesh(core_axis_name="c", subcore_axis_name="s")

_sc_params = pltpu.CompilerParams(
    needs_layout_passes=False, use_tc_tiling_on_sc=False)


@functools.partial(
    pl.kernel,
    out_type=jax.ShapeDtypeStruct((NC * N, D2), jnp.float32),
    mesh=_mesh,
    scratch_types=[
        [pltpu.VMEM((3, K), jnp.int32)] * NBUF,
        [pltpu.VMEM((K, DP), jnp.int32)] * NBUF,
        [pltpu.VMEM((K, D2), jnp.float32)] * NBUF,
        pltpu.VMEM((ZR, D2), jnp.float32),
        pltpu.VMEM_SHARED((N, D2), jnp.float32),
        [pltpu.SemaphoreType.DMA] * NBUF,
        [pltpu.SemaphoreType.DMA] * NBUF,
        [pltpu.SemaphoreType.DMA] * NBUF,
    ],
    compiler_params=_sc_params,
)
def _sc_aggregate(x_hbm, eidx_hbm, out_hbm,
                  ibuf, gbuf, rows, zbuf, acc, isem, gsem, ssem):
    c = lax.axis_index("c")
    s = lax.axis_index("s")

    zv = jnp.zeros((L,), jnp.float32)

    @pl.loop(0, ZR)
    def _(i):
        for j in range(D2 // L):
            zbuf[i, pl.ds(j * L, L)] = zv

    @pl.loop(0, RPW // ZR)
    def _(t):
        pltpu.sync_copy(zbuf, acc.at[pl.ds(s * RPW + t * ZR, ZR)])

    @pl.when(s == 0)
    def _():
        pltpu.sync_copy(zbuf.at[pl.ds(0, TAIL)], acc.at[pl.ds(NS * RPW, TAIL)])

    plsc.subcore_barrier()

    coff = jnp.full((L,), c * N, jnp.int32)
    mask_hi = jnp.full((L,), -65536, jnp.int32)

    def _load_idx(g, slot):
        pltpu.async_copy(eidx_hbm.at[s, g], ibuf[slot], isem[slot])

    def _start_gather(g, slot):
        pltpu.make_async_copy(eidx_hbm.at[s, g], ibuf[slot],
                              isem[slot]).wait()
        for t in range(K // L):
            sl = (0, pl.ds(t * L, L))
            ibuf[slot][sl] = ibuf[slot][sl] + coff
        pltpu.async_copy(x_hbm.at[ibuf[slot].at[0]], gbuf[slot], gsem[slot])

    def _scale(g, slot):
        pltpu.make_async_copy(x_hbm.at[ibuf[slot].at[0]], gbuf[slot],
                              gsem[slot]).wait()
        src = gbuf[slot]
        dst = rows[slot]

        @pl.loop(0, K // L)
        def _(q):
            val16 = plsc.bitcast(ibuf[slot][2, pl.ds(q * L, L)], jnp.float32)
            for l in range(L):
                v = jnp.full((L,), val16[l])
                e = q * L + l
                for p in range(DP // L):
                    w = src[e, pl.ds(p * L, L)]
                    lo = plsc.bitcast(w << 16, jnp.float32)
                    hi = plsc.bitcast(w & mask_hi, jnp.float32)
                    dst[e, pl.ds(p * 2 * L, L)] = lo * v
                    dst[e, pl.ds((p * 2 + 1) * L, L)] = hi * v

        pltpu.async_copy(dst, acc.at[ibuf[slot].at[1]], ssem[slot], add=True)

    def _wait_scatter(g, slot):
        pltpu.make_async_copy(rows[slot], acc.at[ibuf[slot].at[1]],
                              ssem[slot]).wait()

    for g in range(3):
        _load_idx(g, g)
    _start_gather(0, 0)
    _start_gather(1, 1)

    @pl.loop(0, CHUNKS // NBUF)
    def _(h):
        g0 = h * NBUF
        for i in range(NBUF):
            g = g0 + i
            _scale(g, i)
            j3 = (i + 3) % NBUF

            @pl.when(g >= 5)
            def _():
                _wait_scatter(g - 5, j3)

            @pl.when(g + 3 < CHUNKS)
            def _():
                _load_idx(g + 3, j3)

            @pl.when(g + 2 < CHUNKS)
            def _():
                _start_gather(g + 2, (i + 2) % NBUF)

    for g in range(CHUNKS - 5, CHUNKS):
        _wait_scatter(g, g % NBUF)

    plsc.subcore_barrier()

    pltpu.sync_copy(acc.at[pl.ds(s * RPW, RPW)],
                    out_hbm.at[pl.ds(c * N + s * RPW, RPW)])

    @pl.when(s == 0)
    def _():
        pltpu.sync_copy(acc.at[pl.ds(NS * RPW, TAIL)],
                        out_hbm.at[pl.ds(c * N + NS * RPW, TAIL)])


BN = 1000
NB = N // BN


def _mm_relu_body(p0_ref, p1_ref, w_ref, o_ref):
    wv = w_ref[...]
    y = lax.dot_general(p0_ref[...], wv[:D2], (((1,), (0,)), ((), ())),
                        preferred_element_type=jnp.float32,
                        precision=lax.Precision.HIGHEST)
    y += lax.dot_general(p1_ref[...], wv[D2:], (((1,), (0,)), ((), ())),
                         preferred_element_type=jnp.float32,
                         precision=lax.Precision.HIGHEST)
    h = jnp.maximum(y, 0.0).astype(jnp.bfloat16)
    o_ref[0] = h[:, :D2]
    o_ref[1] = h[:, D2:]


def _tc_mm_relu(p, w):
    return pl.pallas_call(
        _mm_relu_body,
        grid=(NB,),
        in_specs=[
            pl.BlockSpec((BN, D2), lambda i: (i, 0)),
            pl.BlockSpec((BN, D2), lambda i: (i + NB, 0)),
            pl.BlockSpec((D, D), lambda i: (0, 0)),
        ],
        out_specs=pl.BlockSpec((NC, BN, D2), lambda i: (0, i, 0)),
        out_shape=jax.ShapeDtypeStruct((NC, N, D2), jnp.bfloat16),
    )(p, p, w)


def _final_body(p0_ref, p1_ref, w_ref, c_ref, o_ref):
    wv = w_ref[...]
    y = lax.dot_general(p0_ref[...], wv[:D2], (((1,), (0,)), ((), ())),
                        preferred_element_type=jnp.float32,
                        precision=lax.Precision.HIGHEST)
    y += lax.dot_general(p1_ref[...], wv[D2:], (((1,), (0,)), ((), ())),
                         preferred_element_type=jnp.float32,
                         precision=lax.Precision.HIGHEST)
    h = jnp.maximum(y, 0.0)
    o_ref[...] = lax.dot_general(h, c_ref[...], (((1,), (0,)), ((), ())),
                                 preferred_element_type=jnp.float32,
                                 precision=lax.Precision.HIGHEST)


def _tc_final(p, w, cls):
    return pl.pallas_call(
        _final_body,
        grid=(NB,),
        in_specs=[
            pl.BlockSpec((BN, D2), lambda i: (i, 0)),
            pl.BlockSpec((BN, D2), lambda i: (i + NB, 0)),
            pl.BlockSpec((D, D), lambda i: (0, 0)),
            pl.BlockSpec((D, D), lambda i: (0, 0)),
        ],
        out_specs=pl.BlockSpec((BN, D), lambda i: (i, 0)),
        out_shape=jax.ShapeDtypeStruct((N, D), jnp.float32),
    )(p, p, w, cls)


def _pack_bf16(a):
    ab = a.astype(jnp.bfloat16).reshape(a.shape[0], DP, 2)
    return lax.bitcast_convert_type(ab, jnp.int32)


def kernel(x, edge_index, edge_val, w0, w1, classifier):
    pad = ((0, 0), (0, EPP - EPS))
    src = jnp.pad(edge_index[0].reshape(NS, EPS), pad)
    dst = jnp.pad(edge_index[1].reshape(NS, EPS), pad)
    val = lax.bitcast_convert_type(jnp.pad(edge_val.reshape(NS, EPS), pad),
                                   jnp.int32)
    eidx = jnp.stack([src.reshape(NS, CHUNKS, K),
                      dst.reshape(NS, CHUNKS, K),
                      val.reshape(NS, CHUNKS, K)], axis=2)

    cls_pad = jnp.zeros((D, D), jnp.float32).at[:, :C].set(classifier)
    perm = jnp.asarray(_PERM128)
    w0p = jnp.take(w0, perm, axis=0)
    w1p = jnp.take(w1, perm, axis=0)

    xcat = jnp.concatenate([x[:, :D2], x[:, D2:]], axis=0)

    p1 = _sc_aggregate(_pack_bf16(xcat), eidx)
    h1 = _tc_mm_relu(p1, w0p)
    h1i = lax.bitcast_convert_type(
        h1.reshape(NC * N, DP, 2), jnp.int32)
    p2 = _sc_aggregate(h1i, eidx)
    out = _tc_final(p2, w1p, cls_pad)
    return out[:, :C]


# speedup vs baseline: 1.2609x; 1.2609x over previous
import functools

import jax
import jax.numpy as jnp
import numpy as np
from jax import lax
from jax.experimental import pallas as pl
from jax.experimental.pallas import tpu as pltpu
from jax.experimental.pallas import tpu_sc as plsc

N = 10000
E = 320000
D = 128
C = 40

NC = 2
NS = 16
L = 16
D2 = D // NC
DP = D2 // 2
EPS = E // NS
K = 80
NBUF = 8
CHUNKS = 256
EPP = CHUNKS * K
RPW = 624
TAIL = N - NS * RPW
ZR = 48

_PERM = np.concatenate([np.arange(0, 32, 2), np.arange(1, 32, 2),
                        32 + np.arange(0, 32, 2), 33 + np.arange(0, 32, 2)])
_PERM128 = np.concatenate([_PERM, 64 + _PERM])

_mesh = plsc.VectorSubcoreMesh(core_axis_name="c", subcore_axis_name="s")

_sc_params = pltpu.CompilerParams(
    needs_layout_passes=False, use_tc_tiling_on_sc=False)


@functools.partial(
    pl.kernel,
    out_type=jax.ShapeDtypeStruct((NC * N, D2), jnp.float32),
    mesh=_mesh,
    scratch_types=[
        [pltpu.VMEM((3, K), jnp.int32)] * NBUF,
        [pltpu.VMEM((K, DP), jnp.int32)] * NBUF,
        [pltpu.VMEM((K, D2), jnp.float32)] * NBUF,
        pltpu.VMEM((ZR, D2), jnp.float32),
        pltpu.VMEM_SHARED((N, D2), jnp.float32),
        [pltpu.SemaphoreType.DMA] * NBUF,
        [pltpu.SemaphoreType.DMA] * NBUF,
        [pltpu.SemaphoreType.DMA] * NBUF,
    ],
    compiler_params=_sc_params,
)
def _sc_aggregate(x_hbm, eidx_hbm, out_hbm,
                  ibuf, gbuf, rows, zbuf, acc, isem, gsem, ssem):
    c = lax.axis_index("c")
    s = lax.axis_index("s")

    zv = jnp.zeros((L,), jnp.float32)

    @pl.loop(0, ZR)
    def _(i):
        for j in range(D2 // L):
            zbuf[i, pl.ds(j * L, L)] = zv

    @pl.loop(0, RPW // ZR)
    def _(t):
        pltpu.sync_copy(zbuf, acc.at[pl.ds(s * RPW + t * ZR, ZR)])

    @pl.when(s == 0)
    def _():
        pltpu.sync_copy(zbuf.at[pl.ds(0, TAIL)], acc.at[pl.ds(NS * RPW, TAIL)])

    plsc.subcore_barrier()

    coff = jnp.full((L,), c * N, jnp.int32)
    mask_hi = jnp.full((L,), -65536, jnp.int32)

    def _load_idx(g, slot):
        pltpu.async_copy(eidx_hbm.at[s, g], ibuf[slot], isem[slot])

    def _start_gather(g, slot):
        pltpu.make_async_copy(eidx_hbm.at[s, g], ibuf[slot],
                              isem[slot]).wait()
        for t in range(K // L):
            sl = (0, pl.ds(t * L, L))
            ibuf[slot][sl] = ibuf[slot][sl] + coff
        pltpu.async_copy(x_hbm.at[ibuf[slot].at[0]], gbuf[slot], gsem[slot])

    def _scale(g, slot):
        pltpu.make_async_copy(x_hbm.at[ibuf[slot].at[0]], gbuf[slot],
                              gsem[slot]).wait()
        src = gbuf[slot]
        dst = rows[slot]

        @pl.loop(0, K // L, unroll=K // L)
        def _(q):
            val16 = plsc.bitcast(ibuf[slot][2, pl.ds(q * L, L)], jnp.float32)
            for l in range(L):
                v = jnp.full((L,), val16[l])
                e = q * L + l
                for p in range(DP // L):
                    w = src[e, pl.ds(p * L, L)]
                    lo = plsc.bitcast(w << 16, jnp.float32)
                    hi = plsc.bitcast(w & mask_hi, jnp.float32)
                    dst[e, pl.ds(p * 2 * L, L)] = lo * v
                    dst[e, pl.ds((p * 2 + 1) * L, L)] = hi * v

        pltpu.async_copy(dst, acc.at[ibuf[slot].at[1]], ssem[slot], add=True)

    def _wait_scatter(g, slot):
        pltpu.make_async_copy(rows[slot], acc.at[ibuf[slot].at[1]],
                              ssem[slot]).wait()

    for g in range(3):
        _load_idx(g, g)
    _start_gather(0, 0)
    _start_gather(1, 1)

    @pl.loop(0, CHUNKS // NBUF)
    def _(h):
        g0 = h * NBUF
        for i in range(NBUF):
            g = g0 + i
            _scale(g, i)
            j3 = (i + 3) % NBUF

            @pl.when(g >= 5)
            def _():
                _wait_scatter(g - 5, j3)

            @pl.when(g + 3 < CHUNKS)
            def _():
                _load_idx(g + 3, j3)

            @pl.when(g + 2 < CHUNKS)
            def _():
                _start_gather(g + 2, (i + 2) % NBUF)

    for g in range(CHUNKS - 5, CHUNKS):
        _wait_scatter(g, g % NBUF)

    plsc.subcore_barrier()

    pltpu.sync_copy(acc.at[pl.ds(s * RPW, RPW)],
                    out_hbm.at[pl.ds(c * N + s * RPW, RPW)])

    @pl.when(s == 0)
    def _():
        pltpu.sync_copy(acc.at[pl.ds(NS * RPW, TAIL)],
                        out_hbm.at[pl.ds(c * N + NS * RPW, TAIL)])


BN = 1000
NB = N // BN


def _mm_relu_body(p0_ref, p1_ref, w_ref, o_ref):
    wv = w_ref[...]
    y = lax.dot_general(p0_ref[...], wv[:D2], (((1,), (0,)), ((), ())),
                        preferred_element_type=jnp.float32,
                        precision=lax.Precision.HIGHEST)
    y += lax.dot_general(p1_ref[...], wv[D2:], (((1,), (0,)), ((), ())),
                         preferred_element_type=jnp.float32,
                         precision=lax.Precision.HIGHEST)
    h = jnp.maximum(y, 0.0).astype(jnp.bfloat16)
    o_ref[0] = h[:, :D2]
    o_ref[1] = h[:, D2:]


def _tc_mm_relu(p, w):
    return pl.pallas_call(
        _mm_relu_body,
        grid=(NB,),
        in_specs=[
            pl.BlockSpec((BN, D2), lambda i: (i, 0)),
            pl.BlockSpec((BN, D2), lambda i: (i + NB, 0)),
            pl.BlockSpec((D, D), lambda i: (0, 0)),
        ],
        out_specs=pl.BlockSpec((NC, BN, D2), lambda i: (0, i, 0)),
        out_shape=jax.ShapeDtypeStruct((NC, N, D2), jnp.bfloat16),
    )(p, p, w)


def _final_body(p0_ref, p1_ref, w_ref, c_ref, o_ref):
    wv = w_ref[...]
    y = lax.dot_general(p0_ref[...], wv[:D2], (((1,), (0,)), ((), ())),
                        preferred_element_type=jnp.float32,
                        precision=lax.Precision.HIGHEST)
    y += lax.dot_general(p1_ref[...], wv[D2:], (((1,), (0,)), ((), ())),
                         preferred_element_type=jnp.float32,
                         precision=lax.Precision.HIGHEST)
    h = jnp.maximum(y, 0.0)
    o_ref[...] = lax.dot_general(h, c_ref[...], (((1,), (0,)), ((), ())),
                                 preferred_element_type=jnp.float32,
                                 precision=lax.Precision.HIGHEST)


def _tc_final(p, w, cls):
    return pl.pallas_call(
        _final_body,
        grid=(NB,),
        in_specs=[
            pl.BlockSpec((BN, D2), lambda i: (i, 0)),
            pl.BlockSpec((BN, D2), lambda i: (i + NB, 0)),
            pl.BlockSpec((D, D), lambda i: (0, 0)),
            pl.BlockSpec((D, D), lambda i: (0, 0)),
        ],
        out_specs=pl.BlockSpec((BN, D), lambda i: (i, 0)),
        out_shape=jax.ShapeDtypeStruct((N, D), jnp.float32),
    )(p, p, w, cls)


def _pack_bf16(a):
    ab = a.astype(jnp.bfloat16).reshape(a.shape[0], DP, 2)
    return lax.bitcast_convert_type(ab, jnp.int32)


def kernel(x, edge_index, edge_val, w0, w1, classifier):
    pad = ((0, 0), (0, EPP - EPS))
    src = jnp.pad(edge_index[0].reshape(NS, EPS), pad)
    dst = jnp.pad(edge_index[1].reshape(NS, EPS), pad)
    val = lax.bitcast_convert_type(jnp.pad(edge_val.reshape(NS, EPS), pad),
                                   jnp.int32)
    eidx = jnp.stack([src.reshape(NS, CHUNKS, K),
                      dst.reshape(NS, CHUNKS, K),
                      val.reshape(NS, CHUNKS, K)], axis=2)

    cls_pad = jnp.zeros((D, D), jnp.float32).at[:, :C].set(classifier)
    perm = jnp.asarray(_PERM128)
    w0p = jnp.take(w0, perm, axis=0)
    w1p = jnp.take(w1, perm, axis=0)

    xcat = jnp.concatenate([x[:, :D2], x[:, D2:]], axis=0)

    p1 = _sc_aggregate(_pack_bf16(xcat), eidx)
    h1 = _tc_mm_relu(p1, w0p)
    h1i = lax.bitcast_convert_type(
        h1.reshape(NC * N, DP, 2), jnp.int32)
    p2 = _sc_aggregate(h1i, eidx)
    out = _tc_final(p2, w1p, cls_pad)
    return out[:, :C]
